# Initial kernel scaffold; baseline (speedup 1.0000x reference)
import jax
import jax.numpy as jnp
from jax.experimental import pallas as pl


def kernel(X_Node, X_Neis, dg_list, emb, W_xi, b_xi, W_rou, b_rou, W_out, b_out):
    raise NotImplementedError("write your pallas kernel here")


# speedup vs baseline: 3.6562x; 3.6562x over previous
import functools

import jax
import jax.numpy as jnp
import numpy as np
from jax import lax
from jax.experimental import pallas as pl
from jax.experimental.pallas import tpu as pltpu
from jax.experimental.pallas import tpu_sc as plsc

V = 4096
F = 128
S = 16
MU = 0.9
E = 32768

NC = 2
NS = 16
NW = NC * NS
EPW = E // NW
CH = 128
NCHUNK = EPW // CH

_MESH = plsc.VectorSubcoreMesh(core_axis_name="c", subcore_axis_name="s",
                               num_cores=NC, num_subcores=NS)


@functools.partial(
    pl.kernel,
    out_type=(
        jax.ShapeDtypeStruct((E, F), jnp.float32),
        jax.ShapeDtypeStruct((E, F), jnp.float32),
        jax.ShapeDtypeStruct((NC, V, F), jnp.float32),
    ),
    mesh=_MESH,
    scratch_types=(
        pltpu.VMEM((CH,), jnp.int32),
        pltpu.VMEM((CH,), jnp.int32),
        pltpu.VMEM((CH, F), jnp.float32),
        pltpu.VMEM((CH, F), jnp.float32),
        pltpu.VMEM((CH, F), jnp.float32),
        pltpu.VMEM_SHARED((V, F), jnp.float32),
        pltpu.SemaphoreType.DMA,
        pltpu.SemaphoreType.DMA,
    ),
)
def _k1_gather_count(emb_hbm, xn_hbm, xm_hbm, zeros_hbm, ones_hbm,
                     xnout_hbm, xmout_hbm, cpart_hbm,
                     idxn_v, idxm_v, rown_v, rowm_v, ones_v, csh, sem1, sem2):
    c = lax.axis_index("c")
    s = lax.axis_index("s")
    wid = s * NC + c
    base = wid * EPW

    @pl.when(s == 0)
    def _init():
        pltpu.sync_copy(zeros_hbm, csh)

    pltpu.sync_copy(ones_hbm, ones_v)
    plsc.subcore_barrier()

    for ch in range(NCHUNK):
        off = base + ch * CH
        pltpu.sync_copy(xn_hbm.at[pl.ds(off, CH)], idxn_v)
        pltpu.sync_copy(xm_hbm.at[pl.ds(off, CH)], idxm_v)
        cpn = pltpu.async_copy(emb_hbm.at[idxn_v], rown_v, sem1)
        cpm = pltpu.async_copy(emb_hbm.at[idxm_v], rowm_v, sem2)
        cpn.wait()
        cpm.wait()
        pltpu.sync_copy(rown_v, xnout_hbm.at[pl.ds(off, CH)])
        pltpu.sync_copy(rowm_v, xmout_hbm.at[pl.ds(off, CH)])
        pltpu.sync_copy(ones_v, csh.at[idxn_v], add=True)

    plsc.subcore_barrier()

    @pl.when(s == 0)
    def _writeback():
        pltpu.sync_copy(csh, cpart_hbm.at[c])


@functools.partial(
    pl.kernel,
    out_type=jax.ShapeDtypeStruct((E, F), jnp.float32),
    mesh=_MESH,
    scratch_types=(
        pltpu.VMEM((CH,), jnp.int32),
        pltpu.VMEM((CH, F), jnp.float32),
        pltpu.SemaphoreType.DMA,
    ),
)
def _k3_gather_states(s1p_hbm, xm_hbm, h_hbm, idx_v, rows_v, sem):
    c = lax.axis_index("c")
    s = lax.axis_index("s")
    base = (s * NC + c) * EPW
    for ch in range(NCHUNK):
        off = base + ch * CH
        pltpu.sync_copy(xm_hbm.at[pl.ds(off, CH)], idx_v)
        pltpu.async_copy(s1p_hbm.at[idx_v], rows_v, sem).wait()
        pltpu.sync_copy(rows_v, h_hbm.at[pl.ds(off, CH)])


@functools.partial(
    pl.kernel,
    out_type=jax.ShapeDtypeStruct((NC, V, F), jnp.float32),
    mesh=_MESH,
    scratch_types=(
        pltpu.VMEM((CH,), jnp.int32),
        pltpu.VMEM((CH, F), jnp.float32),
        pltpu.VMEM_SHARED((V, F), jnp.float32),
    ),
)
def _k5_scatter_add(hn_hbm, xn_hbm, zeros_hbm, qpart_hbm, idx_v, vals_v, qsh):
    c = lax.axis_index("c")
    s = lax.axis_index("s")
    base = (s * NC + c) * EPW

    @pl.when(s == 0)
    def _init():
        pltpu.sync_copy(zeros_hbm, qsh)

    plsc.subcore_barrier()

    for ch in range(NCHUNK):
        off = base + ch * CH
        pltpu.sync_copy(xn_hbm.at[pl.ds(off, CH)], idx_v)
        pltpu.sync_copy(hn_hbm.at[pl.ds(off, CH)], vals_v)
        pltpu.sync_copy(vals_v, qsh.at[idx_v], add=True)

    plsc.subcore_barrier()

    @pl.when(s == 0)
    def _writeback():
        pltpu.sync_copy(qsh, qpart_hbm.at[c])


def _k2_body(emb_ref, wrt_ref, brou_ref, cp0_ref, cp1_ref, s1p_ref):
    beta = jnp.tanh(
        jnp.dot(emb_ref[...], wrt_ref[...], preferred_element_type=jnp.float32)
        + brou_ref[...])
    counts = cp0_ref[...] + cp1_ref[...]
    s1 = beta * counts[:, 0:1]
    s1p_ref[...] = jnp.concatenate(
        [s1, jnp.zeros((V, F - S), jnp.float32)], axis=1)


EB = 1024


def _k4_body(xn_ref, xm_ref, h_ref, dg_ref, w1t_ref, w2t_ref, bxi_ref,
             tsel_ref, gsel_ref, hn_ref):
    a = jnp.tanh(
        jnp.dot(xn_ref[...], w1t_ref[...], preferred_element_type=jnp.float32)
        + jnp.dot(xm_ref[...], w2t_ref[...], preferred_element_type=jnp.float32)
        + bxi_ref[...])
    ht = jnp.dot(h_ref[...], tsel_ref[...], preferred_element_type=jnp.float32)
    hn = jnp.dot(a * ht, gsel_ref[...], preferred_element_type=jnp.float32)
    hn_ref[...] = hn * ((MU / S) / dg_ref[...])


def _k6_body(emb_ref, q0_ref, q1_ref, s1p_ref, woet_ref, wost_ref, bout_ref,
             out_ref):
    s2 = q0_ref[...][:, :S] + q1_ref[...][:, :S] + s1p_ref[:, :S]
    logits = (
        jnp.dot(emb_ref[...], woet_ref[...], preferred_element_type=jnp.float32)
        + jnp.dot(s2, wost_ref[...], preferred_element_type=jnp.float32)
        + bout_ref[...])
    z = logits - jnp.max(logits, axis=1, keepdims=True)
    ez = jnp.exp(z)
    out_ref[...] = ez / jnp.sum(ez, axis=1, keepdims=True)


def _selection_matrices():
    tsel = np.zeros((F, S * S), np.float32)
    gsel = np.zeros((S * S, F), np.float32)
    for i in range(S):
        for j in range(S):
            tsel[j, S * i + j] = 1.0
            gsel[S * i + j, i] = 1.0
    return jnp.asarray(tsel), jnp.asarray(gsel)


def kernel(X_Node, X_Neis, dg_list, emb, W_xi, b_xi, W_rou, b_rou, W_out,
           b_out):
    xn = X_Node.astype(jnp.int32)
    xm = X_Neis.astype(jnp.int32)
    emb = emb.astype(jnp.float32)
    zeros_vs = jnp.zeros((V, F), jnp.float32)
    ones_ch = jnp.ones((CH, F), jnp.float32)

    xn_rows, xm_rows, cpart = _k1_gather_count(emb, xn, xm, zeros_vs, ones_ch)

    w_rou_t = W_rou.T
    brou2 = b_rou.reshape(1, S)
    s1p = pl.pallas_call(
        _k2_body,
        out_shape=jax.ShapeDtypeStruct((V, F), jnp.float32),
    )(emb, w_rou_t, brou2, cpart[0], cpart[1])

    h = _k3_gather_states(s1p, xm)

    tsel, gsel = _selection_matrices()
    w1t = W_xi[:, :F].T
    w2t = W_xi[:, F:].T
    bxi2 = b_xi.reshape(1, S * S)
    dg2 = dg_list.reshape(E, 1).astype(jnp.float32)
    grid = E // EB
    hn = pl.pallas_call(
        _k4_body,
        grid=(grid,),
        in_specs=[
            pl.BlockSpec((EB, F), lambda i: (i, 0)),
            pl.BlockSpec((EB, F), lambda i: (i, 0)),
            pl.BlockSpec((EB, F), lambda i: (i, 0)),
            pl.BlockSpec((EB, 1), lambda i: (i, 0)),
            pl.BlockSpec((F, S * S), lambda i: (0, 0)),
            pl.BlockSpec((F, S * S), lambda i: (0, 0)),
            pl.BlockSpec((1, S * S), lambda i: (0, 0)),
            pl.BlockSpec((F, S * S), lambda i: (0, 0)),
            pl.BlockSpec((S * S, F), lambda i: (0, 0)),
        ],
        out_specs=pl.BlockSpec((EB, F), lambda i: (i, 0)),
        out_shape=jax.ShapeDtypeStruct((E, F), jnp.float32),
    )(xn_rows, xm_rows, h, dg2, w1t, w2t, bxi2, tsel, gsel)

    qpart = _k5_scatter_add(hn, xn, zeros_vs)

    woet = W_out[:, :F].T
    wost = W_out[:, F:].T
    bout2 = b_out.reshape(1, 3)
    out = pl.pallas_call(
        _k6_body,
        out_shape=jax.ShapeDtypeStruct((V, 3), jnp.float32),
    )(emb, qpart[0], qpart[1], s1p, woet, wost, bout2)
    return out


# speedup vs baseline: 4.2799x; 1.1706x over previous
import functools

import jax
import jax.numpy as jnp
import numpy as np
from jax import lax
from jax.experimental import pallas as pl
from jax.experimental.pallas import tpu as pltpu
from jax.experimental.pallas import tpu_sc as plsc

V = 4096
F = 128
S = 16
MU = 0.9
E = 32768

NC = 2
NS = 16
NW = NC * NS
EPW = E // NW
CH = 128
NCHUNK = EPW // CH

_MESH = plsc.VectorSubcoreMesh(core_axis_name="c", subcore_axis_name="s",
                               num_cores=NC, num_subcores=NS)


@functools.partial(
    pl.kernel,
    out_type=(
        jax.ShapeDtypeStruct((E, F), jnp.float32),
        jax.ShapeDtypeStruct((E, F), jnp.float32),
        jax.ShapeDtypeStruct((NC, V, F), jnp.float32),
    ),
    mesh=_MESH,
    scratch_types=(
        pltpu.VMEM((EPW,), jnp.int32),
        pltpu.VMEM((EPW,), jnp.int32),
        pltpu.VMEM((CH,), jnp.int32),
        pltpu.VMEM((2, CH, F), jnp.float32),
        pltpu.VMEM((2, CH, F), jnp.float32),
        pltpu.VMEM((CH, F), jnp.float32),
        pltpu.VMEM_SHARED((V, F), jnp.float32),
        pltpu.SemaphoreType.DMA,
        pltpu.SemaphoreType.DMA,
        pltpu.SemaphoreType.DMA,
        pltpu.SemaphoreType.DMA,
    ),
)
def _k1_gather_count(emb_hbm, xn_hbm, xm_hbm, zeros_hbm, ones_hbm,
                     xnout_hbm, xmout_hbm, cpart_hbm,
                     idxn_v, idxm_v, idxs_v, rown_v, rowm_v, ones_v, csh,
                     semn0, semn1, semm0, semm1):
    c = lax.axis_index("c")
    s = lax.axis_index("s")
    wid = s * NC + c
    base = wid * EPW
    semn = [semn0, semn1]
    semm = [semm0, semm1]

    @pl.when(s == 0)
    def _init():
        pltpu.sync_copy(zeros_hbm, csh)

    pltpu.sync_copy(ones_hbm, ones_v)
    pltpu.sync_copy(xn_hbm.at[pl.ds(base, EPW)], idxn_v)
    pltpu.sync_copy(xm_hbm.at[pl.ds(base, EPW)], idxm_v)

    def start(ch):
        k = ch % 2
        gn = pltpu.async_copy(
            emb_hbm.at[idxn_v.at[pl.ds(ch * CH, CH)]], rown_v.at[k], semn[k])
        gm = pltpu.async_copy(
            emb_hbm.at[idxm_v.at[pl.ds(ch * CH, CH)]], rowm_v.at[k], semm[k])
        return gn, gm

    pend = start(0)
    plsc.subcore_barrier()

    for ch in range(NCHUNK):
        gn, gm = pend
        if ch + 1 < NCHUNK:
            nxt = start(ch + 1)
        off = base + ch * CH
        pltpu.sync_copy(xn_hbm.at[pl.ds(off, CH)], idxs_v)
        pltpu.sync_copy(ones_v, csh.at[idxs_v], add=True)
        gn.wait()
        gm.wait()
        k = ch % 2
        pltpu.sync_copy(rown_v.at[k], xnout_hbm.at[pl.ds(off, CH)])
        pltpu.sync_copy(rowm_v.at[k], xmout_hbm.at[pl.ds(off, CH)])
        if ch + 1 < NCHUNK:
            pend = nxt

    plsc.subcore_barrier()

    @pl.when(s == 0)
    def _writeback():
        pltpu.sync_copy(csh, cpart_hbm.at[c])


@functools.partial(
    pl.kernel,
    out_type=jax.ShapeDtypeStruct((E, F), jnp.float32),
    mesh=_MESH,
    scratch_types=(
        pltpu.VMEM((EPW,), jnp.int32),
        pltpu.VMEM((2, CH, F), jnp.float32),
        pltpu.SemaphoreType.DMA,
        pltpu.SemaphoreType.DMA,
    ),
)
def _k3_gather_states(s1p_hbm, xm_hbm, h_hbm, idx_v, rows_v, sem0, sem1):
    c = lax.axis_index("c")
    s = lax.axis_index("s")
    base = (s * NC + c) * EPW
    sems = [sem0, sem1]
    pltpu.sync_copy(xm_hbm.at[pl.ds(base, EPW)], idx_v)

    def start(ch):
        k = ch % 2
        return pltpu.async_copy(
            s1p_hbm.at[idx_v.at[pl.ds(ch * CH, CH)]], rows_v.at[k], sems[k])

    pend = start(0)
    for ch in range(NCHUNK):
        g = pend
        if ch + 1 < NCHUNK:
            nxt = start(ch + 1)
        g.wait()
        pltpu.sync_copy(rows_v.at[ch % 2], h_hbm.at[pl.ds(base + ch * CH, CH)])
        if ch + 1 < NCHUNK:
            pend = nxt


@functools.partial(
    pl.kernel,
    out_type=jax.ShapeDtypeStruct((NC, V, F), jnp.float32),
    mesh=_MESH,
    scratch_types=(
        pltpu.VMEM((CH,), jnp.int32),
        pltpu.VMEM((CH,), jnp.int32),
        pltpu.VMEM((2, CH, F), jnp.float32),
        pltpu.VMEM_SHARED((V, F), jnp.float32),
        pltpu.SemaphoreType.DMA,
        pltpu.SemaphoreType.DMA,
    ),
)
def _k5_scatter_add(hn_hbm, xn_hbm, zeros_hbm, qpart_hbm,
                    idx_a, idx_b, vals_v, qsh, sem0, sem1):
    c = lax.axis_index("c")
    s = lax.axis_index("s")
    base = (s * NC + c) * EPW
    sems = [sem0, sem1]
    idxs = [idx_a, idx_b]

    @pl.when(s == 0)
    def _init():
        pltpu.sync_copy(zeros_hbm, qsh)

    plsc.subcore_barrier()

    def start(ch):
        k = ch % 2
        pltpu.sync_copy(xn_hbm.at[pl.ds(base + ch * CH, CH)], idxs[k])
        return pltpu.async_copy(
            hn_hbm.at[pl.ds(base + ch * CH, CH)], vals_v.at[k], sems[k])

    pend = start(0)
    for ch in range(NCHUNK):
        g = pend
        if ch + 1 < NCHUNK:
            nxt = start(ch + 1)
        g.wait()
        k = ch % 2
        pltpu.sync_copy(vals_v.at[k], qsh.at[idxs[k]], add=True)
        if ch + 1 < NCHUNK:
            pend = nxt

    plsc.subcore_barrier()

    @pl.when(s == 0)
    def _writeback():
        pltpu.sync_copy(qsh, qpart_hbm.at[c])


def _k2_body(emb_ref, wrt_ref, brou_ref, cp0_ref, cp1_ref, s1p_ref):
    beta = jnp.tanh(
        jnp.dot(emb_ref[...], wrt_ref[...], preferred_element_type=jnp.float32)
        + brou_ref[...])
    counts = cp0_ref[...] + cp1_ref[...]
    s1 = beta * counts[:, 0:1]
    s1p_ref[...] = jnp.concatenate(
        [s1, jnp.zeros((V, F - S), jnp.float32)], axis=1)


EB = 1024


def _k4_body(xn_ref, xm_ref, h_ref, dg_ref, w1t_ref, w2t_ref, bxi_ref,
             tsel_ref, gsel_ref, hn_ref):
    a = jnp.tanh(
        jnp.dot(xn_ref[...], w1t_ref[...], preferred_element_type=jnp.float32)
        + jnp.dot(xm_ref[...], w2t_ref[...], preferred_element_type=jnp.float32)
        + bxi_ref[...])
    ht = jnp.dot(h_ref[...], tsel_ref[...], preferred_element_type=jnp.float32)
    hn = jnp.dot(a * ht, gsel_ref[...], preferred_element_type=jnp.float32)
    hn_ref[...] = hn * ((MU / S) / dg_ref[...])


def _k6_body(emb_ref, q0_ref, q1_ref, s1p_ref, woet_ref, wost_ref, bout_ref,
             out_ref):
    s2 = q0_ref[...][:, :S] + q1_ref[...][:, :S] + s1p_ref[:, :S]
    logits = (
        jnp.dot(emb_ref[...], woet_ref[...], preferred_element_type=jnp.float32)
        + jnp.dot(s2, wost_ref[...], preferred_element_type=jnp.float32)
        + bout_ref[...])
    z = logits - jnp.max(logits, axis=1, keepdims=True)
    ez = jnp.exp(z)
    out_ref[...] = ez / jnp.sum(ez, axis=1, keepdims=True)


def _selection_matrices():
    tsel = np.zeros((F, S * S), np.float32)
    gsel = np.zeros((S * S, F), np.float32)
    for i in range(S):
        for j in range(S):
            tsel[j, S * i + j] = 1.0
            gsel[S * i + j, i] = 1.0
    return jnp.asarray(tsel), jnp.asarray(gsel)


def kernel(X_Node, X_Neis, dg_list, emb, W_xi, b_xi, W_rou, b_rou, W_out,
           b_out):
    xn = X_Node.astype(jnp.int32)
    xm = X_Neis.astype(jnp.int32)
    emb = emb.astype(jnp.float32)
    zeros_vs = jnp.zeros((V, F), jnp.float32)
    ones_ch = jnp.ones((CH, F), jnp.float32)

    xn_rows, xm_rows, cpart = _k1_gather_count(emb, xn, xm, zeros_vs, ones_ch)

    w_rou_t = W_rou.T
    brou2 = b_rou.reshape(1, S)
    s1p = pl.pallas_call(
        _k2_body,
        out_shape=jax.ShapeDtypeStruct((V, F), jnp.float32),
    )(emb, w_rou_t, brou2, cpart[0], cpart[1])

    h = _k3_gather_states(s1p, xm)

    tsel, gsel = _selection_matrices()
    w1t = W_xi[:, :F].T
    w2t = W_xi[:, F:].T
    bxi2 = b_xi.reshape(1, S * S)
    dg2 = dg_list.reshape(E, 1).astype(jnp.float32)
    grid = E // EB
    hn = pl.pallas_call(
        _k4_body,
        grid=(grid,),
        in_specs=[
            pl.BlockSpec((EB, F), lambda i: (i, 0)),
            pl.BlockSpec((EB, F), lambda i: (i, 0)),
            pl.BlockSpec((EB, F), lambda i: (i, 0)),
            pl.BlockSpec((EB, 1), lambda i: (i, 0)),
            pl.BlockSpec((F, S * S), lambda i: (0, 0)),
            pl.BlockSpec((F, S * S), lambda i: (0, 0)),
            pl.BlockSpec((1, S * S), lambda i: (0, 0)),
            pl.BlockSpec((F, S * S), lambda i: (0, 0)),
            pl.BlockSpec((S * S, F), lambda i: (0, 0)),
        ],
        out_specs=pl.BlockSpec((EB, F), lambda i: (i, 0)),
        out_shape=jax.ShapeDtypeStruct((E, F), jnp.float32),
    )(xn_rows, xm_rows, h, dg2, w1t, w2t, bxi2, tsel, gsel)

    qpart = _k5_scatter_add(hn, xn, zeros_vs)

    woet = W_out[:, :F].T
    wost = W_out[:, F:].T
    bout2 = b_out.reshape(1, 3)
    out = pl.pallas_call(
        _k6_body,
        out_shape=jax.ShapeDtypeStruct((V, 3), jnp.float32),
    )(emb, qpart[0], qpart[1], s1p, woet, wost, bout2)
    return out


# speedup vs baseline: 5.1967x; 1.2142x over previous
import functools

import jax
import jax.numpy as jnp
import numpy as np
from jax import lax
from jax.experimental import pallas as pl
from jax.experimental.pallas import tpu as pltpu
from jax.experimental.pallas import tpu_sc as plsc

V = 4096
F = 128
S = 16
MU = 0.9
E = 32768

NC = 2
NS = 16
NW = NC * NS
EPW = E // NW
CH = 128
NCHUNK = EPW // CH

_MESH = plsc.VectorSubcoreMesh(core_axis_name="c", subcore_axis_name="s",
                               num_cores=NC, num_subcores=NS)


@functools.partial(
    pl.kernel,
    out_type=(
        jax.ShapeDtypeStruct((E, F), jnp.float32),
        jax.ShapeDtypeStruct((E, F), jnp.float32),
        jax.ShapeDtypeStruct((V,), jnp.float32),
        jax.ShapeDtypeStruct((E,), jnp.float32),
    ),
    mesh=_MESH,
    scratch_types=(
        pltpu.VMEM((EPW,), jnp.int32),
        pltpu.VMEM((EPW,), jnp.int32),
        pltpu.VMEM((CH,), jnp.int32),
        pltpu.VMEM((CH,), jnp.int32),
        pltpu.VMEM((2, CH, F), jnp.float32),
        pltpu.VMEM((2, CH, F), jnp.float32),
        pltpu.VMEM((CH,), jnp.float32),
        pltpu.VMEM((CH,), jnp.float32),
        pltpu.VMEM_SHARED((V,), jnp.float32),
        pltpu.SemaphoreType.DMA,
        pltpu.SemaphoreType.DMA,
        pltpu.SemaphoreType.DMA,
        pltpu.SemaphoreType.DMA,
        pltpu.SemaphoreType.DMA,
    ),
)
def _k1_gather_count(emb_hbm, xn_hbm, xm_hbm, zeros1_hbm,
                     xnout_hbm, xmout_hbm, counts_hbm, cm_hbm,
                     idxn_v, idxm_v, idxs_v, idxo_v, rown_v, rowm_v,
                     ones1_v, cm_v, csh1,
                     semn0, semn1, semm0, semm1, semg):
    c = lax.axis_index("c")
    s = lax.axis_index("s")
    wid = s * NC + c
    wido = s * NC + (1 - c)
    base = wid * EPW
    baseo = wido * EPW
    semn = [semn0, semn1]
    semm = [semm0, semm1]

    @pl.when(s == 0)
    def _init():
        pltpu.sync_copy(zeros1_hbm, csh1)

    ones16 = jnp.ones((16,), jnp.float32)
    for i in range(CH // 16):
        ones1_v[pl.ds(i * 16, 16)] = ones16

    pltpu.sync_copy(xn_hbm.at[pl.ds(base, EPW)], idxn_v)
    pltpu.sync_copy(xm_hbm.at[pl.ds(base, EPW)], idxm_v)

    def start(ch):
        k = ch % 2
        gn = pltpu.async_copy(
            emb_hbm.at[idxn_v.at[pl.ds(ch * CH, CH)]], rown_v.at[k], semn[k])
        gm = pltpu.async_copy(
            emb_hbm.at[idxm_v.at[pl.ds(ch * CH, CH)]], rowm_v.at[k], semm[k])
        return gn, gm

    pend = start(0)
    plsc.subcore_barrier()

    for ch in range(NCHUNK):
        gn, gm = pend
        if ch + 1 < NCHUNK:
            nxt = start(ch + 1)
        off = base + ch * CH
        pltpu.sync_copy(xn_hbm.at[pl.ds(off, CH)], idxs_v)
        pltpu.sync_copy(ones1_v, csh1.at[idxs_v], add=True)
        pltpu.sync_copy(xn_hbm.at[pl.ds(baseo + ch * CH, CH)], idxo_v)
        pltpu.sync_copy(ones1_v, csh1.at[idxo_v], add=True)
        gn.wait()
        gm.wait()
        k = ch % 2
        pltpu.sync_copy(rown_v.at[k], xnout_hbm.at[pl.ds(off, CH)])
        pltpu.sync_copy(rowm_v.at[k], xmout_hbm.at[pl.ds(off, CH)])
        if ch + 1 < NCHUNK:
            pend = nxt

    plsc.subcore_barrier()

    for ch in range(NCHUNK):
        pltpu.async_copy(
            csh1.at[idxm_v.at[pl.ds(ch * CH, CH)]], cm_v, semg).wait()
        pltpu.sync_copy(cm_v, cm_hbm.at[pl.ds(base + ch * CH, CH)])

    @pl.when((s == 0) & (c == 0))
    def _writeback():
        pltpu.sync_copy(csh1, counts_hbm)


@functools.partial(
    pl.kernel,
    out_type=jax.ShapeDtypeStruct((NC, V, F), jnp.float32),
    mesh=_MESH,
    scratch_types=(
        pltpu.VMEM((CH,), jnp.int32),
        pltpu.VMEM((CH,), jnp.int32),
        pltpu.VMEM((2, CH, F), jnp.float32),
        pltpu.VMEM_SHARED((V, F), jnp.float32),
        pltpu.SemaphoreType.DMA,
        pltpu.SemaphoreType.DMA,
    ),
)
def _k5_scatter_add(hn_hbm, xn_hbm, zeros_hbm, qpart_hbm,
                    idx_a, idx_b, vals_v, qsh, sem0, sem1):
    c = lax.axis_index("c")
    s = lax.axis_index("s")
    base = (s * NC + c) * EPW
    sems = [sem0, sem1]
    idxs = [idx_a, idx_b]

    @pl.when(s == 0)
    def _init():
        pltpu.sync_copy(zeros_hbm, qsh)

    plsc.subcore_barrier()

    def start(ch):
        k = ch % 2
        pltpu.sync_copy(xn_hbm.at[pl.ds(base + ch * CH, CH)], idxs[k])
        return pltpu.async_copy(
            hn_hbm.at[pl.ds(base + ch * CH, CH)], vals_v.at[k], sems[k])

    pend = start(0)
    for ch in range(NCHUNK):
        g = pend
        if ch + 1 < NCHUNK:
            nxt = start(ch + 1)
        g.wait()
        k = ch % 2
        pltpu.sync_copy(vals_v.at[k], qsh.at[idxs[k]], add=True)
        if ch + 1 < NCHUNK:
            pend = nxt

    plsc.subcore_barrier()

    @pl.when(s == 0)
    def _writeback():
        pltpu.sync_copy(qsh, qpart_hbm.at[c])


EB = 1024


def _k4_body(xn_ref, xm_ref, cm_ref, dg_ref, w1t_ref, w2t_ref, bxi_ref,
             wrt_ref, brou_ref, tsel_ref, gsel_ref, ones11_ref, hn_ref):
    xm = xm_ref[...]
    a = jnp.tanh(
        jnp.dot(xn_ref[...], w1t_ref[...], preferred_element_type=jnp.float32)
        + jnp.dot(xm, w2t_ref[...], preferred_element_type=jnp.float32)
        + bxi_ref[...])
    beta_m = jnp.tanh(
        jnp.dot(xm, wrt_ref[...], preferred_element_type=jnp.float32)
        + brou_ref[...])
    cm_col = lax.dot_general(
        cm_ref[...], ones11_ref[...], (((0,), (0,)), ((), ())),
        preferred_element_type=jnp.float32)
    ht = jnp.dot(beta_m, tsel_ref[...], preferred_element_type=jnp.float32)
    hn = jnp.dot(a * ht, gsel_ref[...], preferred_element_type=jnp.float32)
    hn_ref[...] = hn * (cm_col * (MU / S) / dg_ref[...])


def _k6_body(emb_ref, q0_ref, q1_ref, cnt_ref, ones11_ref, wrt_ref, brou_ref,
             woet_ref, wost_ref, bout_ref, out_ref):
    emb = emb_ref[...]
    beta = jnp.tanh(
        jnp.dot(emb, wrt_ref[...], preferred_element_type=jnp.float32)
        + brou_ref[...])
    counts_col = lax.dot_general(
        cnt_ref[...], ones11_ref[...], (((0,), (0,)), ((), ())),
        preferred_element_type=jnp.float32)
    s2 = q0_ref[...][:, :S] + q1_ref[...][:, :S] + beta * counts_col
    logits = (
        jnp.dot(emb, woet_ref[...], preferred_element_type=jnp.float32)
        + jnp.dot(s2, wost_ref[...], preferred_element_type=jnp.float32)
        + bout_ref[...])
    z = logits - jnp.max(logits, axis=1, keepdims=True)
    ez = jnp.exp(z)
    out_ref[...] = ez / jnp.sum(ez, axis=1, keepdims=True)


def _selection_matrices():
    tsel = np.zeros((S, S * S), np.float32)
    gsel = np.zeros((S * S, F), np.float32)
    for i in range(S):
        for j in range(S):
            tsel[j, S * i + j] = 1.0
            gsel[S * i + j, i] = 1.0
    return jnp.asarray(tsel), jnp.asarray(gsel)


def kernel(X_Node, X_Neis, dg_list, emb, W_xi, b_xi, W_rou, b_rou, W_out,
           b_out):
    xn = X_Node.astype(jnp.int32)
    xm = X_Neis.astype(jnp.int32)
    emb = emb.astype(jnp.float32)
    zeros1 = jnp.zeros((V,), jnp.float32)
    zeros_vf = jnp.zeros((V, F), jnp.float32)
    ones11 = jnp.ones((1, 1), jnp.float32)

    xn_rows, xm_rows, counts, cm = _k1_gather_count(emb, xn, xm, zeros1)

    tsel, gsel = _selection_matrices()
    w1t = W_xi[:, :F].T
    w2t = W_xi[:, F:].T
    bxi2 = b_xi.reshape(1, S * S)
    w_rou_t = W_rou.T
    brou2 = b_rou.reshape(1, S)
    dg2 = dg_list.reshape(E, 1).astype(jnp.float32)
    cm2 = cm.reshape(1, E)
    grid = E // EB
    hn = pl.pallas_call(
        _k4_body,
        grid=(grid,),
        in_specs=[
            pl.BlockSpec((EB, F), lambda i: (i, 0)),
            pl.BlockSpec((EB, F), lambda i: (i, 0)),
            pl.BlockSpec((1, EB), lambda i: (0, i)),
            pl.BlockSpec((EB, 1), lambda i: (i, 0)),
            pl.BlockSpec((F, S * S), lambda i: (0, 0)),
            pl.BlockSpec((F, S * S), lambda i: (0, 0)),
            pl.BlockSpec((1, S * S), lambda i: (0, 0)),
            pl.BlockSpec((F, S), lambda i: (0, 0)),
            pl.BlockSpec((1, S), lambda i: (0, 0)),
            pl.BlockSpec((S, S * S), lambda i: (0, 0)),
            pl.BlockSpec((S * S, F), lambda i: (0, 0)),
            pl.BlockSpec((1, 1), lambda i: (0, 0)),
        ],
        out_specs=pl.BlockSpec((EB, F), lambda i: (i, 0)),
        out_shape=jax.ShapeDtypeStruct((E, F), jnp.float32),
    )(xn_rows, xm_rows, cm2, dg2, w1t, w2t, bxi2, w_rou_t, brou2, tsel, gsel,
      ones11)

    qpart = _k5_scatter_add(hn, xn, zeros_vf)

    woet = W_out[:, :F].T
    wost = W_out[:, F:].T
    bout2 = b_out.reshape(1, 3)
    cnt2 = counts.reshape(1, V)
    out = pl.pallas_call(
        _k6_body,
        out_shape=jax.ShapeDtypeStruct((V, 3), jnp.float32),
    )(emb, qpart[0], qpart[1], cnt2, ones11, w_rou_t, brou2, woet, wost,
      bout2)
    return out


# speedup vs baseline: 5.4686x; 1.0523x over previous
import functools

import jax
import jax.numpy as jnp
import numpy as np
from jax import lax
from jax.experimental import pallas as pl
from jax.experimental.pallas import tpu as pltpu
from jax.experimental.pallas import tpu_sc as plsc

V = 4096
F = 128
S = 16
MU = 0.9
E = 32768

NC = 2
NS = 16
NW = NC * NS
EPW = E // NW
CH = 128
NCHUNK = EPW // CH

_MESH = plsc.VectorSubcoreMesh(core_axis_name="c", subcore_axis_name="s",
                               num_cores=NC, num_subcores=NS)


@functools.partial(
    pl.kernel,
    out_type=(
        jax.ShapeDtypeStruct((E, F), jnp.float32),
        jax.ShapeDtypeStruct((E, F), jnp.float32),
        jax.ShapeDtypeStruct((V,), jnp.float32),
        jax.ShapeDtypeStruct((E,), jnp.float32),
    ),
    mesh=_MESH,
    scratch_types=(
        pltpu.VMEM((EPW,), jnp.int32),
        pltpu.VMEM((EPW,), jnp.int32),
        pltpu.VMEM((CH,), jnp.int32),
        pltpu.VMEM((CH,), jnp.int32),
        pltpu.VMEM((2, CH, F), jnp.float32),
        pltpu.VMEM((2, CH, F), jnp.float32),
        pltpu.VMEM((CH,), jnp.float32),
        pltpu.VMEM((CH,), jnp.float32),
        pltpu.VMEM_SHARED((V,), jnp.float32),
        pltpu.SemaphoreType.DMA,
        pltpu.SemaphoreType.DMA,
        pltpu.SemaphoreType.DMA,
        pltpu.SemaphoreType.DMA,
        pltpu.SemaphoreType.DMA,
    ),
)
def _k1_gather_count(emb_hbm, xn_hbm, xm_hbm, zeros1_hbm,
                     xnout_hbm, xmout_hbm, counts_hbm, cm_hbm,
                     idxn_v, idxm_v, idxs_v, idxo_v, rown_v, rowm_v,
                     ones1_v, cm_v, csh1,
                     semn0, semn1, semm0, semm1, semg):
    c = lax.axis_index("c")
    s = lax.axis_index("s")
    wid = s * NC + c
    wido = s * NC + (1 - c)
    base = wid * EPW
    baseo = wido * EPW
    semn = [semn0, semn1]
    semm = [semm0, semm1]

    @pl.when(s == 0)
    def _init():
        pltpu.sync_copy(zeros1_hbm, csh1)

    ones16 = jnp.ones((16,), jnp.float32)
    for i in range(CH // 16):
        ones1_v[pl.ds(i * 16, 16)] = ones16

    pltpu.sync_copy(xn_hbm.at[pl.ds(base, EPW)], idxn_v)
    pltpu.sync_copy(xm_hbm.at[pl.ds(base, EPW)], idxm_v)

    def start(ch):
        k = ch % 2
        gn = pltpu.async_copy(
            emb_hbm.at[idxn_v.at[pl.ds(ch * CH, CH)]], rown_v.at[k], semn[k])
        gm = pltpu.async_copy(
            emb_hbm.at[idxm_v.at[pl.ds(ch * CH, CH)]], rowm_v.at[k], semm[k])
        return gn, gm

    pend = start(0)
    plsc.subcore_barrier()

    for ch in range(NCHUNK):
        gn, gm = pend
        if ch + 1 < NCHUNK:
            nxt = start(ch + 1)
        off = base + ch * CH
        pltpu.sync_copy(xn_hbm.at[pl.ds(off, CH)], idxs_v)
        pltpu.sync_copy(ones1_v, csh1.at[idxs_v], add=True)
        pltpu.sync_copy(xn_hbm.at[pl.ds(baseo + ch * CH, CH)], idxo_v)
        pltpu.sync_copy(ones1_v, csh1.at[idxo_v], add=True)
        gn.wait()
        gm.wait()
        k = ch % 2
        pltpu.sync_copy(rown_v.at[k], xnout_hbm.at[pl.ds(off, CH)])
        pltpu.sync_copy(rowm_v.at[k], xmout_hbm.at[pl.ds(off, CH)])
        if ch + 1 < NCHUNK:
            pend = nxt

    plsc.subcore_barrier()

    for ch in range(NCHUNK):
        pltpu.async_copy(
            csh1.at[idxm_v.at[pl.ds(ch * CH, CH)]], cm_v, semg).wait()
        pltpu.sync_copy(cm_v, cm_hbm.at[pl.ds(base + ch * CH, CH)])

    @pl.when((s == 0) & (c == 0))
    def _writeback():
        pltpu.sync_copy(csh1, counts_hbm)


@functools.partial(
    pl.kernel,
    out_type=jax.ShapeDtypeStruct((NC, V, F), jnp.float32),
    mesh=_MESH,
    scratch_types=(
        pltpu.VMEM((CH,), jnp.int32),
        pltpu.VMEM((CH,), jnp.int32),
        pltpu.VMEM((2, CH, F), jnp.float32),
        pltpu.VMEM_SHARED((V, F), jnp.float32),
        pltpu.SemaphoreType.DMA,
        pltpu.SemaphoreType.DMA,
    ),
)
def _k5_scatter_add(hn_hbm, xn_hbm, zeros_hbm, qpart_hbm,
                    idx_a, idx_b, vals_v, qsh, sem0, sem1):
    c = lax.axis_index("c")
    s = lax.axis_index("s")
    base = (s * NC + c) * EPW
    sems = [sem0, sem1]
    idxs = [idx_a, idx_b]

    @pl.when(s == 0)
    def _init():
        pltpu.sync_copy(zeros_hbm, qsh)

    plsc.subcore_barrier()

    def start(ch):
        k = ch % 2
        pltpu.sync_copy(xn_hbm.at[pl.ds(base + ch * CH, CH)], idxs[k])
        return pltpu.async_copy(
            hn_hbm.at[pl.ds(base + ch * CH, CH)], vals_v.at[k], sems[k])

    pend = start(0)
    for ch in range(NCHUNK):
        g = pend
        if ch + 1 < NCHUNK:
            nxt = start(ch + 1)
        g.wait()
        k = ch % 2
        pltpu.sync_copy(vals_v.at[k], qsh.at[idxs[k]], add=True)
        if ch + 1 < NCHUNK:
            pend = nxt

    plsc.subcore_barrier()

    @pl.when(s == 0)
    def _writeback():
        pltpu.sync_copy(qsh, qpart_hbm.at[c])


EB = 1024


def _k4_body(xn_ref, xm_ref, cm_ref, dg_ref, w1t_ref, w2t_ref, bxi_ref,
             wrt_ref, brou_ref, tsel_ref, gsel_ref, ones11_ref, hn_ref):
    xm = xm_ref[...]
    a = jnp.tanh(
        jnp.dot(xn_ref[...], w1t_ref[...], preferred_element_type=jnp.float32)
        + jnp.dot(xm, w2t_ref[...], preferred_element_type=jnp.float32)
        + bxi_ref[...])
    beta_m = jnp.tanh(
        jnp.dot(xm, wrt_ref[...], preferred_element_type=jnp.float32)
        + brou_ref[...])
    scale_row = cm_ref[...] * (MU / S) / dg_ref[...]
    scale_col = lax.dot_general(
        scale_row, ones11_ref[...], (((0,), (0,)), ((), ())),
        preferred_element_type=jnp.float32)
    ht = jnp.dot(beta_m, tsel_ref[...], preferred_element_type=jnp.float32)
    hn = jnp.dot(a * ht, gsel_ref[...], preferred_element_type=jnp.float32)
    hn_ref[...] = hn * scale_col


def _k6_body(emb_ref, q0_ref, q1_ref, cnt_ref, ones11_ref, wrt_ref, brou_ref,
             woet_ref, wost_ref, bout_ref, out_ref):
    emb = emb_ref[...]
    beta = jnp.tanh(
        jnp.dot(emb, wrt_ref[...], preferred_element_type=jnp.float32)
        + brou_ref[...])
    counts_col = lax.dot_general(
        cnt_ref[...], ones11_ref[...], (((0,), (0,)), ((), ())),
        preferred_element_type=jnp.float32)
    s2 = q0_ref[...][:, :S] + q1_ref[...][:, :S] + beta * counts_col
    logits = (
        jnp.dot(emb, woet_ref[...], preferred_element_type=jnp.float32)
        + jnp.dot(s2, wost_ref[...], preferred_element_type=jnp.float32)
        + bout_ref[...])
    z = logits - jnp.max(logits, axis=1, keepdims=True)
    ez = jnp.exp(z)
    out_ref[...] = ez / jnp.sum(ez, axis=1, keepdims=True)


def _selection_matrices():
    tsel = np.zeros((S, S * S), np.float32)
    gsel = np.zeros((S * S, F), np.float32)
    for i in range(S):
        for j in range(S):
            tsel[j, S * i + j] = 1.0
            gsel[S * i + j, i] = 1.0
    return jnp.asarray(tsel), jnp.asarray(gsel)


def kernel(X_Node, X_Neis, dg_list, emb, W_xi, b_xi, W_rou, b_rou, W_out,
           b_out):
    xn = X_Node.astype(jnp.int32)
    xm = X_Neis.astype(jnp.int32)
    emb = emb.astype(jnp.float32)
    zeros1 = jnp.zeros((V,), jnp.float32)
    zeros_vf = jnp.zeros((V, F), jnp.float32)
    ones11 = jnp.ones((1, 1), jnp.float32)

    xn_rows, xm_rows, counts, cm = _k1_gather_count(emb, xn, xm, zeros1)

    tsel, gsel = _selection_matrices()
    w1t = W_xi[:, :F].T
    w2t = W_xi[:, F:].T
    bxi2 = b_xi.reshape(1, S * S)
    w_rou_t = W_rou.T
    brou2 = b_rou.reshape(1, S)
    dg2 = dg_list.reshape(1, E).astype(jnp.float32)
    cm2 = cm.reshape(1, E)
    grid = E // EB
    hn = pl.pallas_call(
        _k4_body,
        grid=(grid,),
        in_specs=[
            pl.BlockSpec((EB, F), lambda i: (i, 0)),
            pl.BlockSpec((EB, F), lambda i: (i, 0)),
            pl.BlockSpec((1, EB), lambda i: (0, i)),
            pl.BlockSpec((1, EB), lambda i: (0, i)),
            pl.BlockSpec((F, S * S), lambda i: (0, 0)),
            pl.BlockSpec((F, S * S), lambda i: (0, 0)),
            pl.BlockSpec((1, S * S), lambda i: (0, 0)),
            pl.BlockSpec((F, S), lambda i: (0, 0)),
            pl.BlockSpec((1, S), lambda i: (0, 0)),
            pl.BlockSpec((S, S * S), lambda i: (0, 0)),
            pl.BlockSpec((S * S, F), lambda i: (0, 0)),
            pl.BlockSpec((1, 1), lambda i: (0, 0)),
        ],
        out_specs=pl.BlockSpec((EB, F), lambda i: (i, 0)),
        out_shape=jax.ShapeDtypeStruct((E, F), jnp.float32),
    )(xn_rows, xm_rows, cm2, dg2, w1t, w2t, bxi2, w_rou_t, brou2, tsel, gsel,
      ones11)

    qpart = _k5_scatter_add(hn, xn, zeros_vf)

    woet = W_out[:, :F].T
    wost = W_out[:, F:].T
    bout2 = b_out.reshape(1, 3)
    cnt2 = counts.reshape(1, V)
    out = pl.pallas_call(
        _k6_body,
        out_shape=jax.ShapeDtypeStruct((V, 3), jnp.float32),
    )(emb, qpart[0], qpart[1], cnt2, ones11, w_rou_t, brou2, woet, wost,
      bout2)
    return out


# speedup vs baseline: 5.9883x; 1.0950x over previous
import functools

import jax
import jax.numpy as jnp
import numpy as np
from jax import lax
from jax.experimental import pallas as pl
from jax.experimental.pallas import tpu as pltpu
from jax.experimental.pallas import tpu_sc as plsc

V = 4096
F = 128
S = 16
MU = 0.9
E = 32768

NC = 2
NS = 16
NW = NC * NS
EPW = E // NW
CH = 128
NCHUNK = EPW // CH

_MESH = plsc.VectorSubcoreMesh(core_axis_name="c", subcore_axis_name="s",
                               num_cores=NC, num_subcores=NS)


@functools.partial(
    pl.kernel,
    out_type=(
        jax.ShapeDtypeStruct((E, F), jnp.float32),
        jax.ShapeDtypeStruct((E, F), jnp.float32),
        jax.ShapeDtypeStruct((V,), jnp.float32),
        jax.ShapeDtypeStruct((E,), jnp.float32),
    ),
    mesh=_MESH,
    scratch_types=(
        pltpu.VMEM((EPW,), jnp.int32),
        pltpu.VMEM((EPW,), jnp.int32),
        pltpu.VMEM((2 * NCHUNK, CH), jnp.int32),
        pltpu.VMEM((2, CH, F), jnp.float32),
        pltpu.VMEM((2, CH, F), jnp.float32),
        pltpu.VMEM((CH,), jnp.float32),
        pltpu.VMEM((2, CH), jnp.float32),
        pltpu.VMEM_SHARED((V,), jnp.float32),
        pltpu.SemaphoreType.DMA,
        pltpu.SemaphoreType.DMA,
        pltpu.SemaphoreType.DMA,
        pltpu.SemaphoreType.DMA,
        pltpu.SemaphoreType.DMA,
        pltpu.SemaphoreType.DMA,
        pltpu.SemaphoreType.DMA,
        pltpu.SemaphoreType.DMA,
        pltpu.SemaphoreType.DMA,
    ),
)
def _k1_gather_count(emb_hbm, xn_hbm, xm_hbm, zeros1_hbm,
                     xnout_hbm, xmout_hbm, counts_hbm, cm_hbm,
                     idxn_v, idxm_v, idxall_v, rown_v, rowm_v,
                     ones1_v, cm_v, csh1,
                     semn0, semn1, semm0, semm1, semw0, semw1,
                     semh, semsc, semg):
    c = lax.axis_index("c")
    s = lax.axis_index("s")
    wid = s * NC + c
    wido = s * NC + (1 - c)
    base = wid * EPW
    baseo = wido * EPW
    semn = [semn0, semn1]
    semm = [semm0, semm1]
    semw = [semw0, semw1]

    @pl.when(s == 0)
    def _init():
        pltpu.sync_copy(zeros1_hbm, csh1)

    ones16 = jnp.ones((16,), jnp.float32)
    for i in range(CH // 16):
        ones1_v[pl.ds(i * 16, 16)] = ones16

    pltpu.sync_copy(xn_hbm.at[pl.ds(base, EPW)], idxn_v)
    pltpu.sync_copy(xm_hbm.at[pl.ds(base, EPW)], idxm_v)

    def start(ch):
        k = ch % 2
        gn = pltpu.async_copy(
            emb_hbm.at[idxn_v.at[pl.ds(ch * CH, CH)]], rown_v.at[k], semn[k])
        gm = pltpu.async_copy(
            emb_hbm.at[idxm_v.at[pl.ds(ch * CH, CH)]], rowm_v.at[k], semm[k])
        return gn, gm

    pend = start(0)

    hist_loads = []
    for j in range(NCHUNK):
        hist_loads.append(pltpu.async_copy(
            xn_hbm.at[pl.ds(base + j * CH, CH)], idxall_v.at[j], semh))
    for j in range(NCHUNK):
        hist_loads.append(pltpu.async_copy(
            xn_hbm.at[pl.ds(baseo + j * CH, CH)], idxall_v.at[NCHUNK + j],
            semh))
    for hl in hist_loads:
        hl.wait()
    plsc.subcore_barrier()

    scat = []
    for j in range(2 * NCHUNK):
        scat.append(pltpu.async_copy(
            ones1_v, csh1.at[idxall_v.at[j]], semsc, add=True))

    wpend = [None, None]
    for ch in range(NCHUNK):
        gn, gm = pend
        if ch + 1 < NCHUNK:
            kn = (ch + 1) % 2
            if wpend[kn] is not None:
                for w in wpend[kn]:
                    w.wait()
                wpend[kn] = None
            pend = start(ch + 1)
        gn.wait()
        gm.wait()
        k = ch % 2
        off = base + ch * CH
        wpend[k] = (
            pltpu.async_copy(rown_v.at[k], xnout_hbm.at[pl.ds(off, CH)],
                             semw[k]),
            pltpu.async_copy(rowm_v.at[k], xmout_hbm.at[pl.ds(off, CH)],
                             semw[k]),
        )
    for wp in wpend:
        if wp is not None:
            for w in wp:
                w.wait()
    for sc_copy in scat:
        sc_copy.wait()
    plsc.subcore_barrier()

    def cm_start(ch):
        k = ch % 2
        return pltpu.async_copy(
            csh1.at[idxm_v.at[pl.ds(ch * CH, CH)]], cm_v.at[k], semn[k])

    cpend = cm_start(0)
    for ch in range(NCHUNK):
        g = cpend
        if ch + 1 < NCHUNK:
            nxt = cm_start(ch + 1)
        g.wait()
        pltpu.sync_copy(cm_v.at[ch % 2],
                        cm_hbm.at[pl.ds(base + ch * CH, CH)])
        if ch + 1 < NCHUNK:
            cpend = nxt

    @pl.when((s == 0) & (c == 0))
    def _writeback():
        pltpu.sync_copy(csh1, counts_hbm)


@functools.partial(
    pl.kernel,
    out_type=jax.ShapeDtypeStruct((NC, V, F), jnp.float32),
    mesh=_MESH,
    scratch_types=(
        pltpu.VMEM((CH,), jnp.int32),
        pltpu.VMEM((CH,), jnp.int32),
        pltpu.VMEM((2, CH, F), jnp.float32),
        pltpu.VMEM_SHARED((V, F), jnp.float32),
        pltpu.SemaphoreType.DMA,
        pltpu.SemaphoreType.DMA,
    ),
)
def _k5_scatter_add(hn_hbm, xn_hbm, zeros_hbm, qpart_hbm,
                    idx_a, idx_b, vals_v, qsh, sem0, sem1):
    c = lax.axis_index("c")
    s = lax.axis_index("s")
    base = (s * NC + c) * EPW
    sems = [sem0, sem1]
    idxs = [idx_a, idx_b]

    @pl.when(s == 0)
    def _init():
        pltpu.sync_copy(zeros_hbm, qsh)

    plsc.subcore_barrier()

    def start(ch):
        k = ch % 2
        pltpu.sync_copy(xn_hbm.at[pl.ds(base + ch * CH, CH)], idxs[k])
        return pltpu.async_copy(
            hn_hbm.at[pl.ds(base + ch * CH, CH)], vals_v.at[k], sems[k])

    pend = start(0)
    for ch in range(NCHUNK):
        g = pend
        if ch + 1 < NCHUNK:
            nxt = start(ch + 1)
        g.wait()
        k = ch % 2
        pltpu.sync_copy(vals_v.at[k], qsh.at[idxs[k]], add=True)
        if ch + 1 < NCHUNK:
            pend = nxt

    plsc.subcore_barrier()

    @pl.when(s == 0)
    def _writeback():
        pltpu.sync_copy(qsh, qpart_hbm.at[c])


EB = 2048


def _k4_body(xn_ref, xm_ref, cm_ref, dg_ref, w1t_ref, w2t_ref, bxi_ref,
             wrt_ref, brou_ref, tsel_ref, gsel_ref, ones11_ref, hn_ref):
    xm = xm_ref[...]
    a = jnp.tanh(
        jnp.dot(xn_ref[...], w1t_ref[...], preferred_element_type=jnp.float32)
        + jnp.dot(xm, w2t_ref[...], preferred_element_type=jnp.float32)
        + bxi_ref[...])
    beta_m = jnp.tanh(
        jnp.dot(xm, wrt_ref[...], preferred_element_type=jnp.float32)
        + brou_ref[...])
    scale_row = cm_ref[...] * (MU / S) / dg_ref[...]
    scale_col = lax.dot_general(
        scale_row, ones11_ref[...], (((0,), (0,)), ((), ())),
        preferred_element_type=jnp.float32)
    ht = jnp.dot(beta_m, tsel_ref[...], preferred_element_type=jnp.float32)
    hn = jnp.dot(a * ht, gsel_ref[...], preferred_element_type=jnp.float32)
    hn_ref[...] = hn * scale_col


def _k6_body(emb_ref, q0_ref, q1_ref, cnt_ref, ones11_ref, wrt_ref, brou_ref,
             woet_ref, wost_ref, bout_ref, out_ref):
    emb = emb_ref[...]
    beta = jnp.tanh(
        jnp.dot(emb, wrt_ref[...], preferred_element_type=jnp.float32)
        + brou_ref[...])
    counts_col = lax.dot_general(
        cnt_ref[...], ones11_ref[...], (((0,), (0,)), ((), ())),
        preferred_element_type=jnp.float32)
    s2 = q0_ref[...][:, :S] + q1_ref[...][:, :S] + beta * counts_col
    logits = (
        jnp.dot(emb, woet_ref[...], preferred_element_type=jnp.float32)
        + jnp.dot(s2, wost_ref[...], preferred_element_type=jnp.float32)
        + bout_ref[...])
    z = logits - jnp.max(logits, axis=1, keepdims=True)
    ez = jnp.exp(z)
    out_ref[...] = ez / jnp.sum(ez, axis=1, keepdims=True)


def _selection_matrices():
    tsel = np.zeros((S, S * S), np.float32)
    gsel = np.zeros((S * S, F), np.float32)
    for i in range(S):
        for j in range(S):
            tsel[j, S * i + j] = 1.0
            gsel[S * i + j, i] = 1.0
    return jnp.asarray(tsel), jnp.asarray(gsel)


def kernel(X_Node, X_Neis, dg_list, emb, W_xi, b_xi, W_rou, b_rou, W_out,
           b_out):
    xn = X_Node.astype(jnp.int32)
    xm = X_Neis.astype(jnp.int32)
    emb = emb.astype(jnp.float32)
    zeros1 = jnp.zeros((V,), jnp.float32)
    zeros_vf = jnp.zeros((V, F), jnp.float32)
    ones11 = jnp.ones((1, 1), jnp.float32)

    xn_rows, xm_rows, counts, cm = _k1_gather_count(emb, xn, xm, zeros1)

    tsel, gsel = _selection_matrices()
    w1t = W_xi[:, :F].T
    w2t = W_xi[:, F:].T
    bxi2 = b_xi.reshape(1, S * S)
    w_rou_t = W_rou.T
    brou2 = b_rou.reshape(1, S)
    dg2 = dg_list.reshape(1, E).astype(jnp.float32)
    cm2 = cm.reshape(1, E)
    grid = E // EB
    hn = pl.pallas_call(
        _k4_body,
        grid=(grid,),
        in_specs=[
            pl.BlockSpec((EB, F), lambda i: (i, 0)),
            pl.BlockSpec((EB, F), lambda i: (i, 0)),
            pl.BlockSpec((1, EB), lambda i: (0, i)),
            pl.BlockSpec((1, EB), lambda i: (0, i)),
            pl.BlockSpec((F, S * S), lambda i: (0, 0)),
            pl.BlockSpec((F, S * S), lambda i: (0, 0)),
            pl.BlockSpec((1, S * S), lambda i: (0, 0)),
            pl.BlockSpec((F, S), lambda i: (0, 0)),
            pl.BlockSpec((1, S), lambda i: (0, 0)),
            pl.BlockSpec((S, S * S), lambda i: (0, 0)),
            pl.BlockSpec((S * S, F), lambda i: (0, 0)),
            pl.BlockSpec((1, 1), lambda i: (0, 0)),
        ],
        out_specs=pl.BlockSpec((EB, F), lambda i: (i, 0)),
        out_shape=jax.ShapeDtypeStruct((E, F), jnp.float32),
    )(xn_rows, xm_rows, cm2, dg2, w1t, w2t, bxi2, w_rou_t, brou2, tsel, gsel,
      ones11)

    qpart = _k5_scatter_add(hn, xn, zeros_vf)

    woet = W_out[:, :F].T
    wost = W_out[:, F:].T
    bout2 = b_out.reshape(1, 3)
    cnt2 = counts.reshape(1, V)
    out = pl.pallas_call(
        _k6_body,
        out_shape=jax.ShapeDtypeStruct((V, 3), jnp.float32),
    )(emb, qpart[0], qpart[1], cnt2, ones11, w_rou_t, brou2, woet, wost,
      bout2)
    return out
